# Initial kernel scaffold; baseline (speedup 1.0000x reference)
import jax
import jax.numpy as jnp
from jax.experimental import pallas as pl


def kernel(x, positions):
    raise NotImplementedError("write your pallas kernel here")


# speedup vs baseline: 6.3509x; 6.3509x over previous
import functools

import jax
import jax.numpy as jnp
from jax.experimental import pallas as pl
from jax.experimental.pallas import tpu as pltpu

_C = 512
_PPC = 16
_ITS = 5
_S = 8192
_D = 256
_TS = 1024
_NT = _S // _TS


def _knn_pool_kernel(pa_ref, pt_ref, x_ref, out_ref, dist_ref):
    sub_iota = jax.lax.broadcasted_iota(jnp.int32, (_C, _TS), 0)
    lane_iota = jax.lax.broadcasted_iota(jnp.int32, (_C, _TS), 1)

    def dist_tile(cent, t):
        ptile = pt_ref[0, 0:3, pl.ds(t * _TS, _TS)]
        cent3 = cent[:, 0:3]
        ab = jax.lax.dot_general(
            cent3, ptile, (((1,), (0,)), ((), ())),
            preferred_element_type=jnp.float32)
        px, py, pz = ptile[0:1], ptile[1:2], ptile[2:3]
        cx, cy, cz = cent3[:, 0:1], cent3[:, 1:2], cent3[:, 2:3]
        p2 = (px * px + py * py) + pz * pz
        c2 = (cx * cx + cz * cz) + cy * cy
        return (p2 + c2) - 2.0 * ab

    cent0 = pa_ref[0, 0:_C, :]

    def km_iter(it, cent):
        def tile_body(t, sums):
            d = dist_tile(cent, t)
            m = jnp.min(d, axis=0, keepdims=True)
            aidx = jnp.min(jnp.where(d == m, sub_iota, _C), axis=0,
                           keepdims=True)
            oh = (sub_iota == aidx).astype(jnp.float32)
            pa_t = pa_ref[0, pl.ds(t * _TS, _TS), :]
            return sums + jnp.dot(oh, pa_t,
                                  preferred_element_type=jnp.float32)

        sums = jax.lax.fori_loop(0, _NT, tile_body,
                                 jnp.zeros((_C, 4), jnp.float32))
        cnt = jnp.maximum(sums[:, 3:4], 1.0)
        return sums / cnt

    cent = jax.lax.fori_loop(0, _ITS, km_iter, cent0)

    def fill_tile(t, carry):
        dist_ref[:, pl.ds(t * _TS, _TS)] = dist_tile(cent, t)
        return carry

    jax.lax.fori_loop(0, _NT, fill_tile, 0)

    ppc_iota = jax.lax.broadcasted_iota(jnp.int32, (_C, _PPC), 1)

    def topk_iter(j, idx):
        def min_tile(t, m):
            d = dist_ref[:, pl.ds(t * _TS, _TS)]
            return jnp.minimum(m, jnp.min(d, axis=1, keepdims=True))

        m = jax.lax.fori_loop(0, _NT, min_tile,
                              jnp.full((_C, 1), jnp.inf, jnp.float32))

        def idx_tile(t, gi):
            d = dist_ref[:, pl.ds(t * _TS, _TS)]
            li = jnp.where(d == m, lane_iota + t * _TS, _S)
            return jnp.minimum(gi, jnp.min(li, axis=1, keepdims=True))

        gi = jax.lax.fori_loop(0, _NT, idx_tile,
                               jnp.full((_C, 1), _S, jnp.int32))

        idx = jnp.where(ppc_iota == j, gi, idx)

        def upd_tile(t, carry2):
            sel = (lane_iota + t * _TS) == gi
            d = dist_ref[:, pl.ds(t * _TS, _TS)]
            dist_ref[:, pl.ds(t * _TS, _TS)] = jnp.where(
                sel, jnp.float32(jnp.inf), d)
            return carry2

        jax.lax.fori_loop(0, _NT, upd_tile, 0)
        return idx

    idx = jax.lax.fori_loop(0, _PPC, topk_iter,
                            jnp.zeros((_C, _PPC), jnp.int32))

    def mm_tile(t, acc):
        li = lane_iota + t * _TS
        mm = jnp.zeros((_C, _TS), jnp.float32)
        for j in range(_PPC):
            mm = mm + (li == idx[:, j:j + 1]).astype(jnp.float32)
        xt = x_ref[0, pl.ds(t * _TS, _TS), :]
        return acc + jnp.dot(mm, xt, preferred_element_type=jnp.float32)

    acc = jax.lax.fori_loop(0, _NT, mm_tile,
                            jnp.zeros((_C, _D), jnp.float32))
    out_ref[0] = acc * (1.0 / _PPC)


@jax.jit
def kernel(x, positions):
    B, S, D = x.shape
    ones = jnp.ones((B, S, 1), positions.dtype)
    pa = jnp.concatenate([positions, ones], axis=-1)
    pt = jnp.transpose(pa, (0, 2, 1))

    return pl.pallas_call(
        _knn_pool_kernel,
        grid=(B,),
        in_specs=[
            pl.BlockSpec((1, S, 4), lambda b: (b, 0, 0)),
            pl.BlockSpec((1, 4, S), lambda b: (b, 0, 0)),
            pl.BlockSpec((1, S, D), lambda b: (b, 0, 0)),
        ],
        out_specs=pl.BlockSpec((1, _C, D), lambda b: (b, 0, 0)),
        out_shape=jax.ShapeDtypeStruct((B, _C, D), jnp.float32),
        scratch_shapes=[
            pltpu.VMEM((_C, _S), jnp.float32),
        ],
    )(pa, pt, x)


# speedup vs baseline: 6.6319x; 1.0442x over previous
import functools

import jax
import jax.numpy as jnp
from jax.experimental import pallas as pl
from jax.experimental.pallas import tpu as pltpu

_C = 512
_PPC = 16
_ITS = 5
_S = 8192
_D = 256
_TS = 1024
_NT = _S // _TS


def _knn_pool_kernel(pa_ref, pt_ref, x_ref, out_ref, dist_ref):
    sub_iota = jax.lax.broadcasted_iota(jnp.int32, (_C, _TS), 0)
    lane_iota = jax.lax.broadcasted_iota(jnp.int32, (_C, _TS), 1)

    def dist_tile(cent, t):
        ptile = pt_ref[0, 0:3, pl.ds(t * _TS, _TS)]
        cent3 = cent[:, 0:3]
        ab = jax.lax.dot_general(
            cent3, ptile, (((1,), (0,)), ((), ())),
            preferred_element_type=jnp.float32)
        px, py, pz = ptile[0:1], ptile[1:2], ptile[2:3]
        cx, cy, cz = cent3[:, 0:1], cent3[:, 1:2], cent3[:, 2:3]
        p2 = (px * px + py * py) + pz * pz
        c2 = (cx * cx + cz * cz) + cy * cy
        return (p2 + c2) - 2.0 * ab

    cent0 = pa_ref[0, 0:_C, :]

    def km_iter(it, cent):
        def tile_body(t, sums):
            d = dist_tile(cent, t)
            m = jnp.min(d, axis=0, keepdims=True)
            aidx = jnp.min(jnp.where(d == m, sub_iota, _C), axis=0,
                           keepdims=True)
            oh = (sub_iota == aidx).astype(jnp.float32)
            pa_t = pa_ref[0, pl.ds(t * _TS, _TS), :]
            return sums + jnp.dot(oh, pa_t,
                                  preferred_element_type=jnp.float32)

        sums = jax.lax.fori_loop(0, _NT, tile_body,
                                 jnp.zeros((_C, 4), jnp.float32))
        cnt = jnp.maximum(sums[:, 3:4], 1.0)
        return sums / cnt

    cent = jax.lax.fori_loop(0, _ITS, km_iter, cent0)

    def fill_tile(t, carry):
        dist_ref[:, pl.ds(t * _TS, _TS)] = dist_tile(cent, t)
        return carry

    jax.lax.fori_loop(0, _NT, fill_tile, 0)

    ppc_iota = jax.lax.broadcasted_iota(jnp.int32, (_C, _PPC), 1)

    def topk_iter(j, carry):
        idx, gi_prev = carry

        def scan_tile(t, mc):
            m, gi = mc
            ds_ = pl.ds(t * _TS, _TS)
            li = lane_iota + t * _TS
            d = dist_ref[:, ds_]
            d = jnp.where(li == gi_prev, jnp.float32(jnp.inf), d)
            dist_ref[:, ds_] = d
            mt = jnp.min(d, axis=1, keepdims=True)
            lit = jnp.where(d == mt, li, _S)
            it = jnp.min(lit, axis=1, keepdims=True)
            gi = jnp.where(mt < m, it, gi)
            return (jnp.minimum(m, mt), gi)

        _, gi = jax.lax.fori_loop(
            0, _NT, scan_tile,
            (jnp.full((_C, 1), jnp.inf, jnp.float32),
             jnp.full((_C, 1), _S, jnp.int32)))
        idx = jnp.where(ppc_iota == j, gi, idx)
        return idx, gi

    idx, _ = jax.lax.fori_loop(
        0, _PPC, topk_iter,
        (jnp.zeros((_C, _PPC), jnp.int32),
         jnp.full((_C, 1), -1, jnp.int32)))

    def mm_tile(t, acc):
        li = lane_iota + t * _TS
        mm = jnp.zeros((_C, _TS), jnp.float32)
        for j in range(_PPC):
            mm = mm + (li == idx[:, j:j + 1]).astype(jnp.float32)
        xt = x_ref[0, pl.ds(t * _TS, _TS), :]
        return acc + jnp.dot(mm, xt, preferred_element_type=jnp.float32)

    acc = jax.lax.fori_loop(0, _NT, mm_tile,
                            jnp.zeros((_C, _D), jnp.float32))
    out_ref[0] = acc * (1.0 / _PPC)


@jax.jit
def kernel(x, positions):
    B, S, D = x.shape
    ones = jnp.ones((B, S, 1), positions.dtype)
    pa = jnp.concatenate([positions, ones], axis=-1)
    pt = jnp.transpose(pa, (0, 2, 1))

    return pl.pallas_call(
        _knn_pool_kernel,
        grid=(B,),
        in_specs=[
            pl.BlockSpec((1, S, 4), lambda b: (b, 0, 0)),
            pl.BlockSpec((1, 4, S), lambda b: (b, 0, 0)),
            pl.BlockSpec((1, S, D), lambda b: (b, 0, 0)),
        ],
        out_specs=pl.BlockSpec((1, _C, D), lambda b: (b, 0, 0)),
        out_shape=jax.ShapeDtypeStruct((B, _C, D), jnp.float32),
        scratch_shapes=[
            pltpu.VMEM((_C, _S), jnp.float32),
        ],
    )(pa, pt, x)


# speedup vs baseline: 7.6918x; 1.1598x over previous
import functools

import jax
import jax.numpy as jnp
from jax.experimental import pallas as pl
from jax.experimental.pallas import tpu as pltpu

_C = 512
_PPC = 16
_ITS = 5
_S = 8192
_D = 256
_TS = 1024
_NT = _S // _TS


def _knn_pool_kernel(pa_ref, pt_ref, x_ref, out_ref, dist_ref):
    sub_iota = jax.lax.broadcasted_iota(jnp.int32, (_C, _TS), 0)
    lane_iota = jax.lax.broadcasted_iota(jnp.int32, (_C, _TS), 1)

    def dist_tile(cent, t):
        ptile = pt_ref[0, 0:3, pl.ds(t * _TS, _TS)]
        cent3 = cent[:, 0:3]
        ab = jax.lax.dot_general(
            cent3, ptile, (((1,), (0,)), ((), ())),
            preferred_element_type=jnp.float32)
        px, py, pz = ptile[0:1], ptile[1:2], ptile[2:3]
        cx, cy, cz = cent3[:, 0:1], cent3[:, 1:2], cent3[:, 2:3]
        p2 = (px * px + py * py) + pz * pz
        c2 = (cx * cx + cz * cz) + cy * cy
        return (p2 + c2) - 2.0 * ab

    cent0 = pa_ref[0, 0:_C, :]

    def km_iter(it, cent):
        def tile_body(t, sums):
            d = dist_tile(cent, t)
            m = jnp.min(d, axis=0, keepdims=True)
            aidx = jnp.min(jnp.where(d == m, sub_iota, _C), axis=0,
                           keepdims=True)
            oh = (sub_iota == aidx).astype(jnp.float32)
            pa_t = pa_ref[0, pl.ds(t * _TS, _TS), :]
            return sums + jnp.dot(oh, pa_t,
                                  preferred_element_type=jnp.float32)

        sums = jax.lax.fori_loop(0, _NT, tile_body,
                                 jnp.zeros((_C, 4), jnp.float32))
        cnt = jnp.maximum(sums[:, 3:4], 1.0)
        return sums / cnt

    cent = jax.lax.fori_loop(0, _ITS, km_iter, cent0)

    def fill_tile(t, carry):
        dist_ref[:, pl.ds(t * _TS, _TS)] = dist_tile(cent, t)
        return carry

    jax.lax.fori_loop(0, _NT, fill_tile, 0)

    def topk_iter(j, gi_prev):
        def scan_tile(t, mc):
            m, gi = mc
            ds_ = pl.ds(t * _TS, _TS)
            li = lane_iota + t * _TS
            d = dist_ref[:, ds_]
            d = jnp.where(li == gi_prev, jnp.float32(jnp.inf), d)
            dist_ref[:, ds_] = d
            mt = jnp.min(d, axis=1, keepdims=True)
            lit = jnp.where(d == mt, li, _S)
            it = jnp.min(lit, axis=1, keepdims=True)
            gi = jnp.where(mt < m, it, gi)
            return (jnp.minimum(m, mt), gi)

        _, gi = jax.lax.fori_loop(
            0, _NT, scan_tile,
            (jnp.full((_C, 1), jnp.inf, jnp.float32),
             jnp.full((_C, 1), _S, jnp.int32)))
        return gi

    gi_last = jax.lax.fori_loop(0, _PPC, topk_iter,
                                jnp.full((_C, 1), -1, jnp.int32))

    def mm_tile(t, acc):
        li = lane_iota + t * _TS
        d = dist_ref[:, pl.ds(t * _TS, _TS)]
        mm = ((d == jnp.float32(jnp.inf)) | (li == gi_last)).astype(
            jnp.float32)
        xt = x_ref[0, pl.ds(t * _TS, _TS), :]
        return acc + jnp.dot(mm, xt, preferred_element_type=jnp.float32)

    acc = jax.lax.fori_loop(0, _NT, mm_tile,
                            jnp.zeros((_C, _D), jnp.float32))
    out_ref[0] = acc * (1.0 / _PPC)


@jax.jit
def kernel(x, positions):
    B, S, D = x.shape
    ones = jnp.ones((B, S, 1), positions.dtype)
    pa = jnp.concatenate([positions, ones], axis=-1)
    pt = jnp.transpose(pa, (0, 2, 1))

    return pl.pallas_call(
        _knn_pool_kernel,
        grid=(B,),
        in_specs=[
            pl.BlockSpec((1, S, 4), lambda b: (b, 0, 0)),
            pl.BlockSpec((1, 4, S), lambda b: (b, 0, 0)),
            pl.BlockSpec((1, S, D), lambda b: (b, 0, 0)),
        ],
        out_specs=pl.BlockSpec((1, _C, D), lambda b: (b, 0, 0)),
        out_shape=jax.ShapeDtypeStruct((B, _C, D), jnp.float32),
        scratch_shapes=[
            pltpu.VMEM((_C, _S), jnp.float32),
        ],
    )(pa, pt, x)
